# Initial kernel scaffold; baseline (speedup 1.0000x reference)
import jax
import jax.numpy as jnp
from jax.experimental import pallas as pl


def kernel(voxel_features, voxel_coords, params):
    raise NotImplementedError("write your pallas kernel here")


# speedup vs baseline: 1.2029x; 1.2029x over previous
import functools

import jax
import jax.numpy as jnp
import numpy as np
from jax import lax
from jax.experimental import pallas as pl
from jax.experimental.pallas import tpu as pltpu

GRID_Z, GRID_Y, GRID_X = 24, 96, 96
NCELLS = GRID_Z * GRID_Y * GRID_X
MG = 128


def _rup(n, m):
    return -(-n // m) * m


class _Dims:
    def __init__(self, d, h, w):
        self.D, self.H, self.W = d, h, w
        self.S1 = (h + 2) * (w + 2)
        self.S2 = w + 2
        self.LO = _rup(self.S1, 128)
        self.S1m = self.LO + 2 * MG
        self.taps = [(dz, (dy - 1) * self.S2 + (dx - 1))
                     for dz in range(3) for dy in range(3) for dx in range(3)]
        self.ztaps = [(dz, 0) for dz in range(3)]


L1 = _Dims(24, 96, 96)
L2 = _Dims(12, 48, 48)
L3 = _Dims(6, 24, 24)
L4 = _Dims(3, 12, 12)
LOUT = _Dims(2, 24, 24)


def _conv_body(x_ref, w_ref, b_ref, cnt_ref, y_ref, st_ref, *,
               taps, cin, cout, mg, lo):
    zi = pl.program_id(0)
    acc = jnp.zeros((cout, lo), jnp.float32) + b_ref[...]
    dn = (((1,), (0,)), ((), ()))

    def dot(a, b):
        return lax.dot_general(a, b, dn, preferred_element_type=jnp.float32)

    for k, (dz, s) in enumerate(taps):
        xs = x_ref[zi + dz, 0:cin, mg + s:mg + s + lo]
        acc = acc + dot(w_ref[k], xs)
    y_ref[...] = jnp.zeros_like(y_ref)
    y_ref[0, 0:cout, mg:mg + lo] = acc
    m = (cnt_ref[0, 0:1, mg:mg + lo] > 0).astype(jnp.float32)
    s1 = jnp.sum(acc * m, axis=1, keepdims=True)
    s2 = jnp.sum(acc * acc * m, axis=1, keepdims=True)
    sc = jnp.sum(m)

    @pl.when(zi == 0)
    def _():
        st_ref[...] = jnp.zeros_like(st_ref)

    st_ref[:, 0:1] += s1
    st_ref[:, 1:2] += s2
    st_ref[:, 2:3] += sc * jnp.ones((cout, 1), jnp.float32)


def _stats_body(y_ref, cnt_ref, st_ref, *, cout, mg, lo):
    zi = pl.program_id(0)
    y = y_ref[0, 0:cout, mg:mg + lo]
    m = (cnt_ref[0, 0:1, mg:mg + lo] > 0).astype(jnp.float32)
    s1 = jnp.sum(y * m, axis=1, keepdims=True)
    s2 = jnp.sum(y * y * m, axis=1, keepdims=True)
    sc = jnp.sum(m)

    @pl.when(zi == 0)
    def _():
        st_ref[...] = jnp.zeros_like(st_ref)

    st_ref[:, 0:1] += s1
    st_ref[:, 1:2] += s2
    st_ref[:, 2:3] += sc * jnp.ones((cout, 1), jnp.float32)


def _fin_body(*refs, cout, mg, lo, has_skip, has_post):
    y_ref, cnt_ref, ms_ref = refs[0], refs[1], refs[2]
    i = 3
    skip_ref = refs[i] if has_skip else None
    i += has_skip
    post_ref = refs[i] if has_post else None
    i += has_post
    z_ref = refs[i]
    y = y_ref[0, 0:cout, mg:mg + lo]
    z = (y - ms_ref[:, 0:1]) * ms_ref[:, 1:2]
    if has_skip:
        z = z + skip_ref[0, 0:cout, mg:mg + lo]
    m = (cnt_ref[0, 0:1, mg:mg + lo] > 0).astype(jnp.float32)
    z = jnp.maximum(z, 0.0) * m
    if has_post:
        z = z + post_ref[0, 0:cout, mg:mg + lo]
    z_ref[...] = jnp.zeros_like(z_ref)
    z_ref[0, 0:cout, mg:mg + lo] = z


def _full_spec(shape):
    nd = len(shape)
    return pl.BlockSpec(shape, lambda i: (0,) * nd)


def _zoff(arr, d):
    return (arr.shape[0] - d.D) // 2


def _plane_spec(arr, d):
    off = _zoff(arr, d)
    return pl.BlockSpec((1,) + arr.shape[1:], lambda i: (i + off, 0, 0))


def _conv_call(X, w_t, bias, cnt, d, taps=None):
    taps = d.taps if taps is None else taps
    k, cout, cin = w_t.shape
    coutp = _rup(cout, 8)
    y, st = pl.pallas_call(
        functools.partial(_conv_body, taps=taps, cin=cin, cout=cout,
                          mg=MG, lo=d.LO),
        grid=(d.D,),
        in_specs=[
            _full_spec(X.shape),
            _full_spec(w_t.shape),
            _full_spec(bias.shape),
            _plane_spec(cnt, d),
        ],
        out_specs=[
            pl.BlockSpec((1, coutp, d.S1m), lambda i: (i, 0, 0)),
            pl.BlockSpec((cout, 128), lambda i: (0, 0)),
        ],
        out_shape=[
            jax.ShapeDtypeStruct((d.D, coutp, d.S1m), jnp.float32),
            jax.ShapeDtypeStruct((cout, 128), jnp.float32),
        ],
    )(X, w_t, bias, cnt)
    return y, st


def _stats_call(Y, cnt, cout, d):
    return pl.pallas_call(
        functools.partial(_stats_body, cout=cout, mg=MG, lo=d.LO),
        grid=(d.D,),
        in_specs=[_plane_spec(Y, d), _plane_spec(cnt, d)],
        out_specs=pl.BlockSpec((cout, 128), lambda i: (0, 0)),
        out_shape=jax.ShapeDtypeStruct((cout, 128), jnp.float32),
    )(Y, cnt)


def _bn_pack(st):
    cout = st.shape[0]
    cnt = jnp.maximum(st[0, 2], 1.0)
    mu = st[:, 0] / cnt
    var = st[:, 1] / cnt - mu * mu
    isig = 1.0 / jnp.sqrt(var + 1e-3)
    ms = jnp.stack([mu, isig], axis=1)
    return jnp.pad(ms, ((0, 0), (0, 126)))


def _fin_call(Y, cnt, ms, cout, d, skip=None, post=None):
    coutp = _rup(cout, 8)
    ins = [Y, cnt, ms]
    specs = [_plane_spec(Y, d), _plane_spec(cnt, d), _full_spec(ms.shape)]
    if skip is not None:
        ins.append(skip)
        specs.append(_plane_spec(skip, d))
    if post is not None:
        ins.append(post)
        specs.append(_plane_spec(post, d))
    return pl.pallas_call(
        functools.partial(_fin_body, cout=cout, mg=MG, lo=d.LO,
                          has_skip=skip is not None, has_post=post is not None),
        grid=(d.D,),
        in_specs=specs,
        out_specs=pl.BlockSpec((1, coutp, d.S1m), lambda i: (i, 0, 0)),
        out_shape=jax.ShapeDtypeStruct((d.D, coutp, d.S1m), jnp.float32),
    )(*ins)


def _wt(w):
    k = w.shape[0] * w.shape[1] * w.shape[2]
    return w.reshape(k, w.shape[3], w.shape[4]).transpose(0, 2, 1)


def _wt_cnt(w):
    ws = _wt(w)
    k, cout, cin = ws.shape
    top = jnp.concatenate([ws, jnp.zeros((k, cout, 1), jnp.float32)], axis=2)
    bot = jnp.concatenate([jnp.zeros((k, 1, cin), jnp.float32),
                           jnp.ones((k, 1, 1), jnp.float32)], axis=2)
    return jnp.concatenate([top, bot], axis=1)


def _pack_padded(vol, d):
    dd, c = vol.shape[0], vol.shape[1]
    flat = vol.reshape(dd, c, d.S1)
    return jnp.pad(flat, ((0, 0), (0, _rup(c, 8) - c),
                          (MG, d.S1m - MG - d.S1)))


def _pack(vol, d):
    return _pack_padded(
        jnp.pad(vol, ((1, 1), (0, 0), (1, 1), (1, 1))), d)


def _unpack(Y, d, c):
    return Y[:, 0:c, MG:MG + d.S1].reshape(Y.shape[0], c, d.H + 2, d.W + 2)


def _padz(Y):
    return jnp.pad(Y, ((1, 1), (0, 0), (0, 0)))


def _subm(X, cnt, w, d, bias=None):
    ws = _wt(w)
    cout = ws.shape[1]
    b = jnp.zeros((cout, 1), jnp.float32) if bias is None else bias.reshape(cout, 1)
    Y, st = _conv_call(X, ws, b, cnt, d)
    return _padz(_fin_call(Y, cnt, _bn_pack(st), cout, d))


def _sbb(X, cnt, p, d):
    ws1 = _wt(p['w1'])
    cout = ws1.shape[1]
    Y1, st1 = _conv_call(X, ws1, p['b1'].reshape(cout, 1), cnt, d)
    H1 = _padz(_fin_call(Y1, cnt, _bn_pack(st1), cout, d))
    Y2, st2 = _conv_call(H1, _wt(p['w2']), p['b2'].reshape(cout, 1), cnt, d)
    return _padz(_fin_call(Y2, cnt, _bn_pack(st2), cout, d, skip=X))


def _down(X, cnt, w, din, dout):
    cin, cout = w.shape[3], w.shape[4]
    wsc = _wt_cnt(w)
    Xc = jnp.concatenate([X[:, 0:cin], cnt[:, 0:1]], axis=1)
    Xc = jnp.pad(Xc, ((0, 0), (0, _rup(cin + 1, 8) - cin - 1), (0, 0)))
    b = jnp.zeros((cout + 1, 1), jnp.float32)
    Yf, _ = _conv_call(Xc, wsc, b, cnt, din)
    g = _unpack(Yf, din, cout + 1)
    sub = g[0:2 * dout.D:2, :, 1:1 + 2 * dout.H:2, 1:1 + 2 * dout.W:2]
    packed = _pack(sub, dout)
    Y2 = packed[1:-1, 0:cout]
    cnt2 = jnp.pad(packed[:, cout:cout + 1], ((0, 0), (0, 7), (0, 0)))
    st = _stats_call(Y2, cnt2, cout, dout)
    Z = _fin_call(Y2, cnt2, _bn_pack(st), cout, dout)
    return _padz(Z), cnt2


def _inverse(X4, cnt3, w, d4, d3, post):
    cin, cout = w.shape[3], w.shape[4]
    wf = _wt(w[::-1, ::-1, ::-1, :, :])
    a = _unpack(X4, d4, cin)[1:-1, :, 1:1 + d4.H, 1:1 + d4.W]

    def inter(arr, axis):
        z = jnp.zeros_like(arr)
        st = jnp.stack([arr, z], axis=axis + 1)
        shp = list(arr.shape)
        shp[axis] *= 2
        return st.reshape(shp).take(np.arange(2 * arr.shape[axis] - 1),
                                    axis=axis)

    b = inter(inter(inter(a, 0), 2), 3)
    bp = jnp.pad(b, ((1, 2), (0, 0), (1, 2), (1, 2)))
    Xd = _pack_padded(bp, d3)
    Y, st = _conv_call(Xd, wf, jnp.zeros((cout, 1), jnp.float32), cnt3, d3)
    return _fin_call(Y, cnt3, _bn_pack(st), cout, d3, post=post)


def _scatter_dense(vf, coords):
    z = coords[:, 1].astype(jnp.int32)
    y = coords[:, 2].astype(jnp.int32)
    x = coords[:, 3].astype(jnp.int32)
    idx = (z * GRID_Y + y) * GRID_X + x
    n = vf.shape[0]
    payload = jnp.concatenate([vf, jnp.ones((n, 1), jnp.float32)], axis=1)
    return jnp.zeros((NCELLS, 6), jnp.float32).at[idx].add(payload)


def kernel(voxel_features, voxel_coords, params):
    grid = _scatter_dense(voxel_features, voxel_coords)
    g = grid.reshape(GRID_Z, GRID_Y, GRID_X, 6).transpose(0, 3, 1, 2)
    X0 = _pack(g[:, 0:5], L1)
    cnt1 = _pack(g[:, 5:6], L1)

    x = _subm(X0, cnt1, params['subm1'], L1)
    x = _sbb(x, cnt1, params['res1_a'], L1)
    x = _sbb(x, cnt1, params['res1_b'], L1)

    x, cnt2 = _down(x, cnt1, params['spconv2'], L1, L2)
    x = _sbb(x, cnt2, params['res2_a'], L2)
    x = _sbb(x, cnt2, params['res2_b'], L2)

    x, cnt3 = _down(x, cnt2, params['spconv3'], L2, L3)
    x = _sbb(x, cnt3, params['res3_a'], L3)
    x = _sbb(x, cnt3, params['res3_b'], L3)

    p1 = _sbb(x, cnt3, params['p1_stage1'], L3)
    p2, cnt4 = _down(p1, cnt3, params['spconv4'], L3, L4)
    p1 = _subm(p1, cnt3, params['p1_stage1_t11'], L3)
    p1 = _sbb(p1, cnt3, params['p1_stage2'], L3)
    p2 = _sbb(p2, cnt4, params['p2_stage2'], L4)
    p1 = _subm(p1, cnt3, params['p1_stage2_t11'], L3)
    p2 = _subm(p2, cnt4, params['p2_stage2_t22'], L4)
    p1 = _sbb(p1, cnt3, params['p1_stage3'], L3)
    p2 = _sbb(p2, cnt4, params['p2_stage3'], L4)
    p1o = _subm(p1, cnt3, params['p1_conv'], L3)
    p2o = _subm(p2, cnt4, params['p2_conv'], L4)

    a = _subm(p1o, cnt3, params['p1t1'], L3)[1:-1]
    hr = _inverse(p2o, cnt3, params['p2t1'], L4, L3, post=a)

    wco = _wt_cnt(params['conv_out'])
    hrc = jnp.concatenate([_padz(hr)[:, 0:64], cnt3[:, 0:1]], axis=1)
    hrc = jnp.pad(hrc, ((0, 0), (0, 7), (0, 0)))
    Yco, _ = _conv_call(hrc, wco, jnp.zeros((65, 1), jnp.float32), cnt3, L3,
                        taps=L3.ztaps)
    sel = Yco[1:4:2]
    Yo = _padz(sel)
    cnto = _padz(jnp.pad(sel[:, 64:65], ((0, 0), (0, 7), (0, 0))))
    sto = _stats_call(Yo, cnto, 64, LOUT)
    Zo = _fin_call(Yo, cnto, _bn_pack(sto), 64, LOUT)
    out = _unpack(Zo, LOUT, 64)[:, :, 1:25, 1:25]
    return out.transpose(0, 2, 3, 1)[None]
